# baseline (speedup 1.0000x reference)
import jax
import jax.numpy as jnp
from jax.experimental import pallas as pl

HIDDEN = 768
NUM_EXPERTS = 64
TOP_K = 8
ROUTE_SCALE = 2.0
CHUNK_T = 2048
FETCH = 4
SUB_T = 256


def _lexi_tree(v, p):
    rows = v.shape[0]
    while rows > 1:
        h = rows // 2
        av, bv = v[:h], v[h:]
        ap, bp = p[:h], p[h:]
        v = jnp.maximum(av, bv)
        p = jnp.where(bv > av, bp, ap)
        rows = h
    return v, p


def _topk_subtile(logits, bias):
    n = logits.shape[1]
    sc = jax.nn.sigmoid(logits)
    iota_e = jax.lax.broadcasted_iota(jnp.int32, (NUM_EXPERTS, n), 0)
    score_bits = jax.lax.bitcast_convert_type(sc, jnp.int32)
    packed = ((iota_e + 1) << 24) | (score_bits >> 7)
    pf = jax.lax.bitcast_convert_type(packed, jnp.float32)
    work = sc + bias
    vals = []
    idxs = []
    for _ in range(TOP_K):
        _, p = _lexi_tree(work, pf)
        pi = jax.lax.bitcast_convert_type(p, jnp.int32)
        idxs.append((pi >> 24) - 1)
        vals.append((pi & 0x00FFFFFF) << 7)
        work = jnp.where(pf == p, -jnp.inf, work)
    return jnp.concatenate(vals, axis=0), jnp.concatenate(idxs, axis=0)


def _router_kernel(x_ref, w_ref, b_ref, scores_out_ref, idx_out_ref):
    part = pl.program_id(0) % FETCH
    x = x_ref[pl.ds(pl.multiple_of(part * CHUNK_T, CHUNK_T), CHUNK_T), :]
    w = w_ref[:]
    scores = jax.lax.dot_general(
        w, x, (((1,), (1,)), ((), ())), preferred_element_type=jnp.float32
    )
    bias = b_ref[:]
    top_parts = []
    sel_parts = []
    for s in range(0, CHUNK_T, SUB_T):
        tb, se = _topk_subtile(
            jax.lax.slice(scores, (0, s), (NUM_EXPERTS, s + SUB_T)), bias
        )
        top_parts.append(tb)
        sel_parts.append(se)
    top_bits = jnp.concatenate(top_parts, axis=1)
    sel = jnp.concatenate(sel_parts, axis=1)
    top = jax.lax.bitcast_convert_type(top_bits, jnp.float32)
    denom = jnp.sum(top, axis=0, keepdims=True) + 1e-20
    out = top / denom * ROUTE_SCALE
    r = jax.lax.broadcasted_iota(jnp.int32, (TOP_K, TOP_K), 0)
    c = jax.lax.broadcasted_iota(jnp.int32, (TOP_K, TOP_K), 1)
    eye = (r == c).astype(jnp.float32)
    scores_out_ref[:] = jax.lax.dot_general(
        out, eye, (((0,), (0,)), ((), ())), preferred_element_type=jnp.float32
    )
    sel_f = jax.lax.dot_general(
        sel.astype(jnp.float32), eye, (((0,), (0,)), ((), ())),
        preferred_element_type=jnp.float32,
    )
    idx_out_ref[:] = sel_f.astype(jnp.int32)


@jax.jit
def _run(hs, w, bias2d):
    t = hs.shape[0]
    return pl.pallas_call(
        _router_kernel,
        grid=(t // CHUNK_T,),
        in_specs=[
            pl.BlockSpec((FETCH * CHUNK_T, HIDDEN), lambda i: (i // FETCH, 0)),
            pl.BlockSpec((NUM_EXPERTS, HIDDEN), lambda i: (0, 0)),
            pl.BlockSpec((NUM_EXPERTS, 1), lambda i: (0, 0)),
        ],
        out_specs=[
            pl.BlockSpec((CHUNK_T, TOP_K), lambda i: (i, 0)),
            pl.BlockSpec((CHUNK_T, TOP_K), lambda i: (i, 0)),
        ],
        out_shape=[
            jax.ShapeDtypeStruct((t, TOP_K), jnp.float32),
            jax.ShapeDtypeStruct((t, TOP_K), jnp.int32),
        ],
    )(hs, w, bias2d)


def kernel(hidden_states, expert_bias, W):
    hidden_dim = hidden_states.shape[-1]
    hs = hidden_states.reshape(-1, hidden_dim)
    bias2d = expert_bias.reshape(NUM_EXPERTS, 1)
    top_scores, selected_experts = _run(hs, W, bias2d)
    return top_scores, selected_experts


# speedup vs baseline: 1.2909x; 1.2909x over previous
import jax
import jax.numpy as jnp
from jax.experimental import pallas as pl

HIDDEN = 768
NUM_EXPERTS = 64
TOP_K = 8
ROUTE_SCALE = 2.0
CHUNK_T = 4096
SUB_T = 256


def _lexi_tree(v, p):
    rows = v.shape[0]
    while rows > 1:
        h = rows // 2
        av, bv = v[:h], v[h:]
        ap, bp = p[:h], p[h:]
        v = jnp.maximum(av, bv)
        p = jnp.where(bv > av, bp, ap)
        rows = h
    return v, p


def _topk_subtile(logits, bias):
    n = logits.shape[1]
    sc = jax.nn.sigmoid(logits)
    iota_e = jax.lax.broadcasted_iota(jnp.int32, (NUM_EXPERTS, n), 0)
    score_bits = jax.lax.bitcast_convert_type(sc, jnp.int32)
    packed = ((iota_e + 1) << 24) | (score_bits >> 7)
    pf = jax.lax.bitcast_convert_type(packed, jnp.float32)
    work = sc + bias
    vals = []
    idxs = []
    for _ in range(TOP_K):
        _, p = _lexi_tree(work, pf)
        pi = jax.lax.bitcast_convert_type(p, jnp.int32)
        idxs.append((pi >> 24) - 1)
        vals.append((pi & 0x00FFFFFF) << 7)
        work = jnp.where(pf == p, -jnp.inf, work)
    return jnp.concatenate(vals, axis=0), jnp.concatenate(idxs, axis=0)


def _router_kernel(x_ref, w_ref, b_ref, scores_out_ref, idx_out_ref):
    x = x_ref[:]
    w = w_ref[:]
    scores = jax.lax.dot_general(
        w, x, (((1,), (1,)), ((), ())), preferred_element_type=jnp.float32
    )
    bias = b_ref[:]
    top_parts = []
    sel_parts = []
    for s in range(0, CHUNK_T, SUB_T):
        tb, se = _topk_subtile(
            jax.lax.slice(scores, (0, s), (NUM_EXPERTS, s + SUB_T)), bias
        )
        top_parts.append(tb)
        sel_parts.append(se)
    top_bits = jnp.concatenate(top_parts, axis=1)
    sel = jnp.concatenate(sel_parts, axis=1)
    top = jax.lax.bitcast_convert_type(top_bits, jnp.float32)
    denom = jnp.sum(top, axis=0, keepdims=True) + 1e-20
    out = top / denom * ROUTE_SCALE
    r = jax.lax.broadcasted_iota(jnp.int32, (TOP_K, TOP_K), 0)
    c = jax.lax.broadcasted_iota(jnp.int32, (TOP_K, TOP_K), 1)
    eye = (r == c).astype(jnp.float32)
    scores_out_ref[:] = jax.lax.dot_general(
        out, eye, (((0,), (0,)), ((), ())), preferred_element_type=jnp.float32
    )
    sel_f = jax.lax.dot_general(
        sel.astype(jnp.float32), eye, (((0,), (0,)), ((), ())),
        preferred_element_type=jnp.float32,
    )
    idx_out_ref[:] = sel_f.astype(jnp.int32)


@jax.jit
def _run(hs, w, bias2d):
    t = hs.shape[0]
    return pl.pallas_call(
        _router_kernel,
        grid=(t // CHUNK_T,),
        in_specs=[
            pl.BlockSpec((CHUNK_T, HIDDEN), lambda i: (i, 0)),
            pl.BlockSpec((NUM_EXPERTS, HIDDEN), lambda i: (0, 0)),
            pl.BlockSpec((NUM_EXPERTS, 1), lambda i: (0, 0)),
        ],
        out_specs=[
            pl.BlockSpec((CHUNK_T, TOP_K), lambda i: (i, 0)),
            pl.BlockSpec((CHUNK_T, TOP_K), lambda i: (i, 0)),
        ],
        out_shape=[
            jax.ShapeDtypeStruct((t, TOP_K), jnp.float32),
            jax.ShapeDtypeStruct((t, TOP_K), jnp.int32),
        ],
    )(hs, w, bias2d)


def kernel(hidden_states, expert_bias, W):
    hidden_dim = hidden_states.shape[-1]
    hs = hidden_states.reshape(-1, hidden_dim)
    bias2d = expert_bias.reshape(NUM_EXPERTS, 1)
    top_scores, selected_experts = _run(hs, W, bias2d)
    return top_scores, selected_experts
